# baseline (speedup 1.0000x reference)
import functools

import numpy as np
import jax
import jax.numpy as jnp
from jax.experimental import pallas as pl
from jax.experimental.pallas import tpu as pltpu
from jax.experimental.pallas import tpu_sc as plsc

_B, _N, _C, _P, _NS = 16, 1024, 256, 256, 16
_R2 = np.float32(0.3 ** 2)
_DIRF = np.float32(np.pi / 12.0)
_HI = jax.lax.Precision.HIGHEST
_DEF = jax.lax.Precision.DEFAULT


def _vote_body(sf_ref, spt_ref, wv1, bv1, wv2, bv2, wv3o, bv3o, wv3f, bv3f,
               ws1ft, ws1xt, bs1r, vp_ref, qt_ref):
    sf = sf_ref[0]
    spt = spt_ref[0]
    h = jnp.maximum(jnp.dot(wv1[...], sf, precision=_DEF) + bv1[...], 0.0)
    h = jnp.maximum(jnp.dot(wv2[...], h, precision=_DEF) + bv2[...], 0.0)
    off = jnp.dot(wv3o[...], h, precision=_DEF) + bv3o[...]
    feat = jnp.dot(wv3f[...], h, precision=_DEF) + bv3f[...]
    vp = spt + off
    vf = sf + feat
    vp_ref[0] = vp
    qt = jax.lax.dot_general(vf, ws1ft[...], (((0,), (0,)), ((), ())),
                             precision=_DEF)
    qt = qt + jax.lax.dot_general(vp, ws1xt[...], (((0,), (0,)), ((), ())),
                                  precision=_DEF)
    qt_ref[0] = qt + bs1r[...]


def _fps_body(vp_ref, rk_ref, dist_ref, rkv_ref):
    px = vp_ref[:, 0, :]
    py = vp_ref[:, 1, :]
    pz = vp_ref[:, 2, :]
    lanes = jax.lax.broadcasted_iota(jnp.int32, (_B, _N), 1)
    dist_ref[...] = jnp.full((_B, _N), 1e10, jnp.float32)
    rkv_ref[...] = jnp.full((_B, _N), 1 << 30, jnp.int32)
    last0 = jnp.min(lanes, axis=1, keepdims=True)

    def body(i, last):
        mask = lanes == last
        rkv_ref[...] = jnp.where(mask, i, rkv_ref[...])
        lx = jnp.sum(jnp.where(mask, px, 0.0), axis=1, keepdims=True)
        ly = jnp.sum(jnp.where(mask, py, 0.0), axis=1, keepdims=True)
        lz = jnp.sum(jnp.where(mask, pz, 0.0), axis=1, keepdims=True)
        d = ((px - lx) ** 2 + (py - ly) ** 2) + (pz - lz) ** 2
        dist = jnp.minimum(dist_ref[...], d)
        dist_ref[...] = dist
        m = jnp.max(dist, axis=1, keepdims=True)
        return jnp.min(jnp.where(dist == m, lanes, _N), axis=1, keepdims=True)

    last = jax.lax.fori_loop(0, _P - 1, body, last0, unroll=5)
    rk_ref[:, 0, :] = jnp.where(lanes == last, _P - 1, rkv_ref[...])


def _knn_body(boff, vp_ref, rk_ref, selg_ref, nxyz_ref):
    b = pl.program_id(0) + boff
    px = vp_ref[0, 0:1, :]
    py = vp_ref[0, 1:2, :]
    pz = vp_ref[0, 2:3, :]
    rk = rk_ref[0]
    pidx = jax.lax.broadcasted_iota(jnp.int32, (_P, 1), 0)
    S = rk == pidx
    nx = jnp.sum(jnp.where(S, px, 0.0), axis=1, keepdims=True)
    ny = jnp.sum(jnp.where(S, py, 0.0), axis=1, keepdims=True)
    nz = jnp.sum(jnp.where(S, pz, 0.0), axis=1, keepdims=True)
    d2 = ((nx - px) ** 2 + (ny - py) ** 2) + (nz - pz) ** 2

    lanes = jax.lax.broadcasted_iota(jnp.int32, (_P, _N), 1)
    v = d2
    idx0 = None
    cols = []
    for s in range(_NS):
        m = jnp.min(v, axis=1, keepdims=True)
        idx = jnp.min(jnp.where(v == m, lanes, _N), axis=1, keepdims=True)
        if s == 0:
            idx0 = idx
        cols.append(jnp.where(m < _R2, idx, idx0))
        v = jnp.where(lanes == idx, jnp.inf, v)
    selg_ref[0] = jnp.concatenate(cols, axis=1) + b * _N
    nxyz_ref[0] = jnp.concatenate([nx, ny, nz], axis=1)


def _sc_gather_body(rpw, table_hbm, idx_hbm, out_hbm, idxs_v,
                    r0, r1, r2, r3, gsem, wsem):
    wid = jax.lax.axis_index("s") * 2 + jax.lax.axis_index("c")
    base = wid * rpw
    nw = rpw // 512
    pltpu.sync_copy(idx_hbm.at[pl.ds(base, rpw)], idxs_v)
    bufs = (r0, r1, r2, r3)

    def gather(ch, k):
        return pltpu.async_copy(
            table_hbm.at[idxs_v.at[pl.ds(ch * 128, 128)]], bufs[k], gsem)

    cur_g = [gather(k, k) for k in range(4)]
    cur_w = [None] * 4
    for w in range(nw):
        for k in range(4):
            cur_g[k].wait()
            cur_w[k] = pltpu.async_copy(
                bufs[k], out_hbm.at[pl.ds(base + (w * 4 + k) * 128, 128)],
                wsem)
        if w + 1 < nw:
            for k in range(4):
                cur_w[k].wait()
                cur_g[k] = gather((w + 1) * 4 + k, k)
    for k in range(4):
        cur_w[k].wait()


def _sc_gather(table, idx):
    n = idx.shape[0]
    rpw = n // 32
    run = pl.kernel(
        functools.partial(_sc_gather_body, rpw),
        out_type=jax.ShapeDtypeStruct((n, 128), jnp.float32),
        mesh=plsc.VectorSubcoreMesh(core_axis_name="c", subcore_axis_name="s"),
        scratch_types=[
            pltpu.VMEM((rpw,), jnp.int32),
            pltpu.VMEM((128, 128), jnp.float32),
            pltpu.VMEM((128, 128), jnp.float32),
            pltpu.VMEM((128, 128), jnp.float32),
            pltpu.VMEM((128, 128), jnp.float32),
            pltpu.SemaphoreType.DMA,
            pltpu.SemaphoreType.DMA,
        ],
    )
    return run(table, idx)


def _head_body(g_ref, nxyz_ref, ws1xt, ws2t, bs2r, ws3t, bs3r,
               wp1t, bp1r, wp2t, bp2r, wct, bcr, wrt, brr, meanf, out_ref):
    nxyz = nxyz_ref[0]
    nx = nxyz[:, 0:1]
    ny = nxyz[:, 1:2]
    nz = nxyz[:, 2:3]
    w3 = ws1xt[...]
    cx = nx * w3[0:1, :] + ny * w3[1:2, :] + nz * w3[2:3, :]

    rows = []
    for c in range(8):
        gc = g_ref[0, c * 512:(c + 1) * 512, :]
        cxe = jnp.reshape(
            jnp.broadcast_to(
                jnp.reshape(cx[c * 32:(c + 1) * 32, :], (32, 1, 128)),
                (32, _NS, 128)),
            (512, 128))
        z = jnp.maximum(gc - cxe, 0.0)
        z = jnp.maximum(jnp.dot(z, ws2t[...], precision=_DEF) + bs2r[...], 0.0)
        z = jnp.maximum(jnp.dot(z, ws3t[...], precision=_DEF) + bs3r[...], 0.0)
        rows.append(jnp.max(jnp.reshape(z, (32, _NS, 128)), axis=1))
    feats = jnp.concatenate(rows, axis=0)

    h2 = jnp.maximum(jnp.dot(feats, wp1t[...], precision=_DEF) + bp1r[...], 0.0)
    h2 = jnp.maximum(jnp.dot(h2, wp2t[...], precision=_DEF) + bp2r[...], 0.0)
    cls = jnp.dot(h2, wct[...], precision=_DEF) + bcr[...]
    reg = jnp.dot(h2, wrt[...], precision=_DEF) + brr[...]
    center = nxyz + reg[:, 0:3]
    out = jnp.concatenate([
        cls[:, 0:2], cls[:, 2:12], center, reg[:, 3:15],
        reg[:, 15:27] * _DIRF, reg[:, 27:37],
        reg[:, 37:67] * meanf[...],
    ], axis=1)
    out_ref[0] = out


def _full(shape):
    nd = len(shape)
    return pl.BlockSpec(shape, lambda b, _n=nd: (0,) * _n)


def kernel(seed_points, seed_features, seed_indices, Wv1, bv1, Wv2, bv2,
           Wv3, bv3, Ws1, bs1, Ws2, bs2, Ws3, bs3, Wp1, bp1, Wp2, bp2,
           Wc, bc, Wr, br, mean_sizes):
    f32 = jnp.float32
    spt = jnp.transpose(seed_points, (0, 2, 1))
    wv3o, wv3f = Wv3[:3], Wv3[3:]
    bv3o, bv3f = bv3[:3].reshape(3, 1), bv3[3:].reshape(_C, 1)
    ws1xt = jnp.transpose(Ws1[:, :3])
    ws1ft = jnp.transpose(Ws1[:, 3:])

    vp, qt = pl.pallas_call(
        _vote_body,
        grid=(_B,),
        in_specs=[
            pl.BlockSpec((1, _C, _N), lambda b: (b, 0, 0)),
            pl.BlockSpec((1, 3, _N), lambda b: (b, 0, 0)),
            _full((_C, _C)), _full((_C, 1)),
            _full((_C, _C)), _full((_C, 1)),
            _full((3, _C)), _full((3, 1)),
            _full((_C, _C)), _full((_C, 1)),
            _full((_C, 128)), _full((3, 128)), _full((1, 128)),
        ],
        out_specs=[
            pl.BlockSpec((1, 3, _N), lambda b: (b, 0, 0)),
            pl.BlockSpec((1, _N, 128), lambda b: (b, 0, 0)),
        ],
        out_shape=[
            jax.ShapeDtypeStruct((_B, 3, _N), f32),
            jax.ShapeDtypeStruct((_B, _N, 128), f32),
        ],
        compiler_params=pltpu.CompilerParams(
            dimension_semantics=("parallel",)),
    )(seed_features, spt, Wv1, bv1.reshape(_C, 1), Wv2, bv2.reshape(_C, 1),
      wv3o, bv3o, wv3f, bv3f, ws1ft, ws1xt, bs1.reshape(1, 128))

    rk = pl.pallas_call(
        _fps_body,
        out_shape=jax.ShapeDtypeStruct((_B, 1, _N), jnp.int32),
        scratch_shapes=[
            pltpu.VMEM((_B, _N), jnp.float32),
            pltpu.VMEM((_B, _N), jnp.int32),
        ],
    )(vp)

    table = qt.reshape(_B * _N, 128)
    hw = _B // 4

    def knn_half(boff):
        return pl.pallas_call(
            functools.partial(_knn_body, boff),
            grid=(hw,),
            in_specs=[
                pl.BlockSpec((1, 3, _N), lambda b, _o=boff: (b + _o, 0, 0)),
                pl.BlockSpec((1, 1, _N), lambda b, _o=boff: (b + _o, 0, 0)),
            ],
            out_specs=[
                pl.BlockSpec((1, _P, _NS), lambda b: (b, 0, 0)),
                pl.BlockSpec((1, _P, 3), lambda b: (b, 0, 0)),
            ],
            out_shape=[
                jax.ShapeDtypeStruct((hw, _P, _NS), jnp.int32),
                jax.ShapeDtypeStruct((hw, _P, 3), f32),
            ],
            compiler_params=pltpu.CompilerParams(
                dimension_semantics=("arbitrary",)),
        )(vp, rk)

    def head_half(g, nxyz):
        return pl.pallas_call(
            _head_body,
            grid=(hw,),
            in_specs=[
                pl.BlockSpec((1, _P * _NS, 128), lambda b: (b, 0, 0)),
                pl.BlockSpec((1, _P, 3), lambda b: (b, 0, 0)),
                _full((3, 128)),
                _full((128, 128)), _full((1, 128)),
                _full((128, 128)), _full((1, 128)),
                _full((128, 128)), _full((1, 128)),
                _full((128, 128)), _full((1, 128)),
                _full((128, 12)), _full((1, 12)),
                _full((128, 67)), _full((1, 67)),
                _full((1, 30)),
            ],
            out_specs=[pl.BlockSpec((1, _P, 79), lambda b: (b, 0, 0))],
            out_shape=[jax.ShapeDtypeStruct((hw, _P, 79), f32)],
            compiler_params=pltpu.CompilerParams(
                dimension_semantics=("arbitrary",)),
        )(g, nxyz, ws1xt,
          jnp.transpose(Ws2), bs2.reshape(1, 128),
          jnp.transpose(Ws3), bs3.reshape(1, 128),
          jnp.transpose(Wp1), bp1.reshape(1, 128),
          jnp.transpose(Wp2), bp2.reshape(1, 128),
          jnp.transpose(Wc), bc.reshape(1, 12),
          jnp.transpose(Wr), br.reshape(1, 67),
          mean_sizes.reshape(1, 30))[0]

    parts = []
    gs = []
    for q in range(_B // hw):
        selg, nxyz = knn_half(q * hw)
        gs.append((_sc_gather(table, selg.reshape(hw * _P * _NS)), nxyz))
    for g, nxyz in gs:
        parts.append(head_half(g.reshape(hw, _P * _NS, 128), nxyz))
    return jnp.concatenate(parts, axis=0)


# speedup vs baseline: 1.0269x; 1.0269x over previous
import functools

import numpy as np
import jax
import jax.numpy as jnp
from jax.experimental import pallas as pl
from jax.experimental.pallas import tpu as pltpu
from jax.experimental.pallas import tpu_sc as plsc

_B, _N, _C, _P, _NS = 16, 1024, 256, 256, 16
_R2 = np.float32(0.3 ** 2)
_DIRF = np.float32(np.pi / 12.0)
_HI = jax.lax.Precision.HIGHEST
_DEF = jax.lax.Precision.DEFAULT


def _vote_body(sf_ref, spt_ref, wv1, bv1, wv2, bv2, wv3o, bv3o, wv3f, bv3f,
               ws1ft, ws1xt, bs1r, vp_ref, qt_ref):
    sf = sf_ref[0]
    spt = spt_ref[0]
    h = jnp.maximum(jnp.dot(wv1[...], sf, precision=_DEF) + bv1[...], 0.0)
    h = jnp.maximum(jnp.dot(wv2[...], h, precision=_DEF) + bv2[...], 0.0)
    off = jnp.dot(wv3o[...], h, precision=_DEF) + bv3o[...]
    feat = jnp.dot(wv3f[...], h, precision=_DEF) + bv3f[...]
    vp = spt + off
    vf = sf + feat
    vp_ref[0] = vp
    qt = jax.lax.dot_general(vf, ws1ft[...], (((0,), (0,)), ((), ())),
                             precision=_DEF)
    qt = qt + jax.lax.dot_general(vp, ws1xt[...], (((0,), (0,)), ((), ())),
                                  precision=_DEF)
    qt_ref[0] = qt + bs1r[...]


def _fps_body(vp_ref, rk_ref, dist_ref, rkv_ref):
    px = vp_ref[:, 0, :]
    py = vp_ref[:, 1, :]
    pz = vp_ref[:, 2, :]
    lanes = jax.lax.broadcasted_iota(jnp.int32, (_B, _N), 1)
    dist_ref[...] = jnp.full((_B, _N), 1e10, jnp.float32)
    rkv_ref[...] = jnp.full((_B, _N), 1 << 30, jnp.int32)
    last0 = jnp.min(lanes, axis=1, keepdims=True)

    def body(i, last):
        mask = lanes == last
        rkv_ref[...] = jnp.where(mask, i, rkv_ref[...])
        lx = jnp.sum(jnp.where(mask, px, 0.0), axis=1, keepdims=True)
        ly = jnp.sum(jnp.where(mask, py, 0.0), axis=1, keepdims=True)
        lz = jnp.sum(jnp.where(mask, pz, 0.0), axis=1, keepdims=True)
        d = ((px - lx) ** 2 + (py - ly) ** 2) + (pz - lz) ** 2
        dist = jnp.minimum(dist_ref[...], d)
        dist_ref[...] = dist
        m = jnp.max(dist, axis=1, keepdims=True)
        return jnp.min(jnp.where(dist == m, lanes, _N), axis=1, keepdims=True)

    last = jax.lax.fori_loop(0, _P - 1, body, last0, unroll=5)
    rk_ref[:, 0, :] = jnp.where(lanes == last, _P - 1, rkv_ref[...])


def _knn_body(boff, vp_ref, rk_ref, selg_ref, nxyz_ref):
    b = pl.program_id(0) + boff
    px = vp_ref[0, 0:1, :]
    py = vp_ref[0, 1:2, :]
    pz = vp_ref[0, 2:3, :]
    rk = rk_ref[0]
    pidx = jax.lax.broadcasted_iota(jnp.int32, (_P, 1), 0)
    S = rk == pidx
    nx = jnp.sum(jnp.where(S, px, 0.0), axis=1, keepdims=True)
    ny = jnp.sum(jnp.where(S, py, 0.0), axis=1, keepdims=True)
    nz = jnp.sum(jnp.where(S, pz, 0.0), axis=1, keepdims=True)
    d2 = ((nx - px) ** 2 + (ny - py) ** 2) + (nz - pz) ** 2

    lanes = jax.lax.broadcasted_iota(jnp.int32, (_P, _N), 1)
    v = d2
    idx0 = None
    cols = []
    for s in range(_NS):
        m = jnp.min(v, axis=1, keepdims=True)
        idx = jnp.min(jnp.where(v == m, lanes, _N), axis=1, keepdims=True)
        if s == 0:
            idx0 = idx
        cols.append(jnp.where(m < _R2, idx, idx0))
        v = jnp.where(lanes == idx, jnp.inf, v)
    selg_ref[0] = jnp.concatenate(cols, axis=1) + b * _N
    nxyz_ref[0] = jnp.concatenate([nx, ny, nz], axis=1)


def _sc_gather_body(rpw, table_hbm, idx_hbm, out_hbm, idxs_v,
                    r0, r1, r2, r3, gsem, wsem):
    wid = jax.lax.axis_index("s") * 2 + jax.lax.axis_index("c")
    base = wid * rpw
    nw = rpw // 512
    pltpu.sync_copy(idx_hbm.at[pl.ds(base, rpw)], idxs_v)
    bufs = (r0, r1, r2, r3)

    def gather(ch, k):
        return pltpu.async_copy(
            table_hbm.at[idxs_v.at[pl.ds(ch * 128, 128)]], bufs[k], gsem)

    cur_g = [gather(k, k) for k in range(4)]
    cur_w = [None] * 4
    for w in range(nw):
        for k in range(4):
            cur_g[k].wait()
            cur_w[k] = pltpu.async_copy(
                bufs[k], out_hbm.at[pl.ds(base + (w * 4 + k) * 128, 128)],
                wsem)
        if w + 1 < nw:
            for k in range(4):
                cur_w[k].wait()
                cur_g[k] = gather((w + 1) * 4 + k, k)
    for k in range(4):
        cur_w[k].wait()


def _sc_gather(table, idx):
    n = idx.shape[0]
    rpw = n // 32
    run = pl.kernel(
        functools.partial(_sc_gather_body, rpw),
        out_type=jax.ShapeDtypeStruct((n, 128), jnp.float32),
        mesh=plsc.VectorSubcoreMesh(core_axis_name="c", subcore_axis_name="s"),
        scratch_types=[
            pltpu.VMEM((rpw,), jnp.int32),
            pltpu.VMEM((128, 128), jnp.float32),
            pltpu.VMEM((128, 128), jnp.float32),
            pltpu.VMEM((128, 128), jnp.float32),
            pltpu.VMEM((128, 128), jnp.float32),
            pltpu.SemaphoreType.DMA,
            pltpu.SemaphoreType.DMA,
        ],
    )
    return run(table, idx)


def _head_body(g_ref, nxyz_ref, ws1xt, ws2t, bs2r, ws3t, bs3r,
               wp1t, bp1r, wp2t, bp2r, wct, bcr, wrt, brr, meanf, out_ref):
    nxyz = nxyz_ref[0]
    nx = nxyz[:, 0:1]
    ny = nxyz[:, 1:2]
    nz = nxyz[:, 2:3]
    w3 = ws1xt[...]
    cx = nx * w3[0:1, :] + ny * w3[1:2, :] + nz * w3[2:3, :]

    rows = []
    for c in range(8):
        gc = g_ref[0, c * 512:(c + 1) * 512, :]
        cxe = jnp.reshape(
            jnp.broadcast_to(
                jnp.reshape(cx[c * 32:(c + 1) * 32, :], (32, 1, 128)),
                (32, _NS, 128)),
            (512, 128))
        z = jnp.maximum(gc - cxe, 0.0)
        z = jnp.maximum(jnp.dot(z, ws2t[...], precision=_DEF) + bs2r[...], 0.0)
        z = jnp.maximum(jnp.dot(z, ws3t[...], precision=_DEF) + bs3r[...], 0.0)
        rows.append(jnp.max(jnp.reshape(z, (32, _NS, 128)), axis=1))
    feats = jnp.concatenate(rows, axis=0)

    h2 = jnp.maximum(jnp.dot(feats, wp1t[...], precision=_DEF) + bp1r[...], 0.0)
    h2 = jnp.maximum(jnp.dot(h2, wp2t[...], precision=_DEF) + bp2r[...], 0.0)
    cls = jnp.dot(h2, wct[...], precision=_DEF) + bcr[...]
    reg = jnp.dot(h2, wrt[...], precision=_DEF) + brr[...]
    center = nxyz + reg[:, 0:3]
    out = jnp.concatenate([
        cls[:, 0:2], cls[:, 2:12], center, reg[:, 3:15],
        reg[:, 15:27] * _DIRF, reg[:, 27:37],
        reg[:, 37:67] * meanf[...],
    ], axis=1)
    out_ref[0] = out


def _full(shape):
    nd = len(shape)
    return pl.BlockSpec(shape, lambda b, _n=nd: (0,) * _n)


def kernel(seed_points, seed_features, seed_indices, Wv1, bv1, Wv2, bv2,
           Wv3, bv3, Ws1, bs1, Ws2, bs2, Ws3, bs3, Wp1, bp1, Wp2, bp2,
           Wc, bc, Wr, br, mean_sizes):
    f32 = jnp.float32
    spt = jnp.transpose(seed_points, (0, 2, 1))
    wv3o, wv3f = Wv3[:3], Wv3[3:]
    bv3o, bv3f = bv3[:3].reshape(3, 1), bv3[3:].reshape(_C, 1)
    ws1xt = jnp.transpose(Ws1[:, :3])
    ws1ft = jnp.transpose(Ws1[:, 3:])

    vp, qt = pl.pallas_call(
        _vote_body,
        grid=(_B,),
        in_specs=[
            pl.BlockSpec((1, _C, _N), lambda b: (b, 0, 0)),
            pl.BlockSpec((1, 3, _N), lambda b: (b, 0, 0)),
            _full((_C, _C)), _full((_C, 1)),
            _full((_C, _C)), _full((_C, 1)),
            _full((3, _C)), _full((3, 1)),
            _full((_C, _C)), _full((_C, 1)),
            _full((_C, 128)), _full((3, 128)), _full((1, 128)),
        ],
        out_specs=[
            pl.BlockSpec((1, 3, _N), lambda b: (b, 0, 0)),
            pl.BlockSpec((1, _N, 128), lambda b: (b, 0, 0)),
        ],
        out_shape=[
            jax.ShapeDtypeStruct((_B, 3, _N), f32),
            jax.ShapeDtypeStruct((_B, _N, 128), f32),
        ],
        compiler_params=pltpu.CompilerParams(
            dimension_semantics=("parallel",)),
    )(seed_features, spt, Wv1, bv1.reshape(_C, 1), Wv2, bv2.reshape(_C, 1),
      wv3o, bv3o, wv3f, bv3f, ws1ft, ws1xt, bs1.reshape(1, 128))

    rk = pl.pallas_call(
        _fps_body,
        out_shape=jax.ShapeDtypeStruct((_B, 1, _N), jnp.int32),
        scratch_shapes=[
            pltpu.VMEM((_B, _N), jnp.float32),
            pltpu.VMEM((_B, _N), jnp.int32),
        ],
    )(vp)

    table = qt.reshape(_B * _N, 128)
    hw = _B // 2

    def knn_half(boff):
        return pl.pallas_call(
            functools.partial(_knn_body, boff),
            grid=(hw,),
            in_specs=[
                pl.BlockSpec((1, 3, _N), lambda b, _o=boff: (b + _o, 0, 0)),
                pl.BlockSpec((1, 1, _N), lambda b, _o=boff: (b + _o, 0, 0)),
            ],
            out_specs=[
                pl.BlockSpec((1, _P, _NS), lambda b: (b, 0, 0)),
                pl.BlockSpec((1, _P, 3), lambda b: (b, 0, 0)),
            ],
            out_shape=[
                jax.ShapeDtypeStruct((hw, _P, _NS), jnp.int32),
                jax.ShapeDtypeStruct((hw, _P, 3), f32),
            ],
            compiler_params=pltpu.CompilerParams(
                dimension_semantics=("arbitrary",)),
        )(vp, rk)

    def head_half(g, nxyz):
        return pl.pallas_call(
            _head_body,
            grid=(hw,),
            in_specs=[
                pl.BlockSpec((1, _P * _NS, 128), lambda b: (b, 0, 0)),
                pl.BlockSpec((1, _P, 3), lambda b: (b, 0, 0)),
                _full((3, 128)),
                _full((128, 128)), _full((1, 128)),
                _full((128, 128)), _full((1, 128)),
                _full((128, 128)), _full((1, 128)),
                _full((128, 128)), _full((1, 128)),
                _full((128, 12)), _full((1, 12)),
                _full((128, 67)), _full((1, 67)),
                _full((1, 30)),
            ],
            out_specs=[pl.BlockSpec((1, _P, 79), lambda b: (b, 0, 0))],
            out_shape=[jax.ShapeDtypeStruct((hw, _P, 79), f32)],
            compiler_params=pltpu.CompilerParams(
                dimension_semantics=("arbitrary",)),
        )(g, nxyz, ws1xt,
          jnp.transpose(Ws2), bs2.reshape(1, 128),
          jnp.transpose(Ws3), bs3.reshape(1, 128),
          jnp.transpose(Wp1), bp1.reshape(1, 128),
          jnp.transpose(Wp2), bp2.reshape(1, 128),
          jnp.transpose(Wc), bc.reshape(1, 12),
          jnp.transpose(Wr), br.reshape(1, 67),
          mean_sizes.reshape(1, 30))[0]

    selg0, nxyz0 = knn_half(0)
    g0 = _sc_gather(table, selg0.reshape(hw * _P * _NS))
    selg1, nxyz1 = knn_half(hw)
    g1 = _sc_gather(table, selg1.reshape(hw * _P * _NS))
    out0 = head_half(g0.reshape(hw, _P * _NS, 128), nxyz0)
    out1 = head_half(g1.reshape(hw, _P * _NS, 128), nxyz1)
    return jnp.concatenate([out0, out1], axis=0)
